# baseline (speedup 1.0000x reference)
import functools

import jax
import jax.numpy as jnp
from jax import lax
from jax.experimental import pallas as pl
from jax.experimental.pallas import tpu as pltpu
from jax.experimental.pallas import tpu_sc as plsc

N = 1000000
B = 16384
D = 32
PACK = 128 // D
C = 1 << 18
CSHIFT = 18
NC = 2
NS = 16
L = 16
NW = NC * NS
BPW = B // NW
HALF = BPW // 2
G = HALF // L

BX = 512
GRID = C // BX
NBLK = -(-N // BX) - 1


def _repack_body(t0, t1, t2, t3, out_ref):
    m = jnp.concatenate([t0[...], t1[...], t2[...], t3[...]], axis=0)
    eye = jnp.eye(128, dtype=jnp.float32)
    out_ref[...] = lax.dot_general(
        m, eye, (((0,), (0,)), ((), ())),
        preferred_element_type=jnp.float32)


def _in_spec(k):
    return pl.BlockSpec(
        (D, BX), lambda g, k=k: (0, jnp.minimum(g + k * GRID, NBLK)))


_repack = pl.pallas_call(
    _repack_body,
    grid=(GRID,),
    in_specs=[_in_spec(0), _in_spec(1), _in_spec(2), _in_spec(3)],
    out_specs=pl.BlockSpec((BX, 128), lambda g: (g, 0)),
    out_shape=jax.ShapeDtypeStruct((C, 128), jnp.float32),
)

_mesh = plsc.VectorSubcoreMesh(core_axis_name="c", subcore_axis_name="s")


@functools.partial(
    pl.kernel,
    mesh=_mesh,
    out_type=jax.ShapeDtypeStruct((B,), jnp.float32),
    scratch_types=[
        pltpu.VMEM((BPW,), jnp.int32),
        pltpu.VMEM((BPW,), jnp.int32),
        pltpu.VMEM((BPW,), jnp.int32),
        pltpu.VMEM((BPW,), jnp.int32),
        pltpu.VMEM((HALF, 128), jnp.float32),
        pltpu.VMEM((HALF, 128), jnp.float32),
        pltpu.VMEM((BPW,), jnp.float32),
        pltpu.SemaphoreType.DMA,
    ],
    compiler_params=pltpu.CompilerParams(
        needs_layout_passes=False, use_tc_tiling_on_sc=False),
)
def _mf_dot(u_hbm, v_hbm, up_hbm, vp_hbm, out_hbm,
            uraw, vraw, upk, vpk, rows_u, rows_v, outv, sem):
    wid = lax.axis_index("s") * NC + lax.axis_index("c")
    base = wid * BPW

    pltpu.sync_copy(u_hbm.at[pl.ds(base, BPW)], uraw)
    pltpu.sync_copy(v_hbm.at[pl.ds(base, BPW)], vraw)

    def build(t, carry):
        upk[pl.ds(t * L, L)] = uraw[pl.ds(t * L, L)] & (C - 1)
        vpk[pl.ds(t * L, L)] = vraw[pl.ds(t * L, L)] & (C - 1)
        return carry

    lax.fori_loop(0, BPW // L, build, 0)

    lane = lax.iota(jnp.int32, L)

    def half(h, carry):
        hb = h * HALF
        cu = pltpu.async_copy(up_hbm.at[upk.at[pl.ds(hb, HALF)]], rows_u, sem)
        cv = pltpu.async_copy(vp_hbm.at[vpk.at[pl.ds(hb, HALF)]], rows_v, sem)
        cu.wait()
        cv.wait()

        def group(g, carry2):
            row = g * L + lane
            offu = lax.shift_right_logical(uraw[pl.ds(hb + g * L, L)], CSHIFT) * D
            offv = lax.shift_right_logical(vraw[pl.ds(hb + g * L, L)], CSHIFT) * D
            acc = jnp.zeros((L,), jnp.float32)
            for d in range(D):
                xu = plsc.load_gather(rows_u, [row, offu + d])
                xv = plsc.load_gather(rows_v, [row, offv + d])
                acc = acc + xu * xv
            outv[pl.ds(hb + g * L, L)] = acc
            return carry2

        lax.fori_loop(0, G, group, 0)
        return carry

    lax.fori_loop(0, 2, half, 0)

    pltpu.sync_copy(outv, out_hbm.at[pl.ds(base, BPW)])


def kernel(u, v, user_emb, item_emb):
    ut = jnp.transpose(user_emb)
    vt = jnp.transpose(item_emb)
    up = _repack(ut, ut, ut, ut)
    vp = _repack(vt, vt, vt, vt)
    return _mf_dot(u.astype(jnp.int32), v.astype(jnp.int32), up, vp)


# speedup vs baseline: 1.6224x; 1.6224x over previous
import functools

import jax
import jax.numpy as jnp
from jax import lax
from jax.experimental import pallas as pl
from jax.experimental.pallas import tpu as pltpu
from jax.experimental.pallas import tpu_sc as plsc

N = 1000000
B = 16384
D = 32
PACK = 128 // D
C = 1 << 18
CSHIFT = 18
NC = 2
NS = 16
L = 16
NW = NC * NS
BPW = B // NW
HALF = BPW // 2
G = HALF // L

BX = 1024
GRID = C // BX
NBLK = -(-N // BX) - 1


def _repack_body(t0, t1, t2, t3, eye_ref, out_ref):
    m = jnp.concatenate([t0[...], t1[...], t2[...], t3[...]], axis=0)
    out_ref[...] = lax.dot_general(
        m.astype(jnp.bfloat16), eye_ref[...], (((0,), (0,)), ((), ())),
        preferred_element_type=jnp.float32)


def _in_spec(k):
    return pl.BlockSpec(
        (D, BX), lambda g, k=k: (0, jnp.minimum(g + k * GRID, NBLK)))


_repack = pl.pallas_call(
    _repack_body,
    grid=(GRID,),
    in_specs=[_in_spec(0), _in_spec(1), _in_spec(2), _in_spec(3),
              pl.BlockSpec((128, 128), lambda g: (0, 0))],
    out_specs=pl.BlockSpec((BX, 128), lambda g: (g, 0)),
    out_shape=jax.ShapeDtypeStruct((C, 128), jnp.float32),
)

_mesh = plsc.VectorSubcoreMesh(core_axis_name="c", subcore_axis_name="s")


@functools.partial(
    pl.kernel,
    mesh=_mesh,
    out_type=jax.ShapeDtypeStruct((B,), jnp.float32),
    scratch_types=[
        pltpu.VMEM((BPW,), jnp.int32),
        pltpu.VMEM((BPW,), jnp.int32),
        pltpu.VMEM((BPW,), jnp.int32),
        pltpu.VMEM((BPW,), jnp.int32),
        pltpu.VMEM((HALF, 128), jnp.float32),
        pltpu.VMEM((HALF, 128), jnp.float32),
        pltpu.VMEM((BPW,), jnp.float32),
        pltpu.SemaphoreType.DMA,
    ],
    compiler_params=pltpu.CompilerParams(
        needs_layout_passes=False, use_tc_tiling_on_sc=False),
)
def _mf_dot(u_hbm, v_hbm, up_hbm, vp_hbm, out_hbm,
            uraw, vraw, upk, vpk, rows_u, rows_v, outv, sem):
    wid = lax.axis_index("s") * NC + lax.axis_index("c")
    base = wid * BPW

    pltpu.sync_copy(u_hbm.at[pl.ds(base, BPW)], uraw)
    pltpu.sync_copy(v_hbm.at[pl.ds(base, BPW)], vraw)

    def build(t, carry):
        upk[pl.ds(t * L, L)] = uraw[pl.ds(t * L, L)] & (C - 1)
        vpk[pl.ds(t * L, L)] = vraw[pl.ds(t * L, L)] & (C - 1)
        return carry

    lax.fori_loop(0, BPW // L, build, 0)

    lane = lax.iota(jnp.int32, L)

    def half(h, carry):
        hb = h * HALF
        cu = pltpu.async_copy(up_hbm.at[upk.at[pl.ds(hb, HALF)]], rows_u, sem)
        cv = pltpu.async_copy(vp_hbm.at[vpk.at[pl.ds(hb, HALF)]], rows_v, sem)
        cu.wait()
        cv.wait()

        def group(g, carry2):
            row = g * L + lane
            offu = lax.shift_right_logical(uraw[pl.ds(hb + g * L, L)], CSHIFT) * D
            offv = lax.shift_right_logical(vraw[pl.ds(hb + g * L, L)], CSHIFT) * D
            acc = jnp.zeros((L,), jnp.float32)
            for d in range(D):
                xu = plsc.load_gather(rows_u, [row, offu + d])
                xv = plsc.load_gather(rows_v, [row, offv + d])
                acc = acc + xu * xv
            outv[pl.ds(hb + g * L, L)] = acc
            return carry2

        lax.fori_loop(0, G, group, 0)
        return carry

    lax.fori_loop(0, 2, half, 0)

    pltpu.sync_copy(outv, out_hbm.at[pl.ds(base, BPW)])


def kernel(u, v, user_emb, item_emb):
    ut = jnp.transpose(user_emb)
    vt = jnp.transpose(item_emb)
    eye = jnp.eye(128, dtype=jnp.bfloat16)
    up = _repack(ut, ut, ut, ut, eye)
    vp = _repack(vt, vt, vt, vt, eye)
    return _mf_dot(u.astype(jnp.int32), v.astype(jnp.int32), up, vp)


# speedup vs baseline: 1.6376x; 1.0094x over previous
import functools

import jax
import jax.numpy as jnp
from jax import lax
from jax.experimental import pallas as pl
from jax.experimental.pallas import tpu as pltpu
from jax.experimental.pallas import tpu_sc as plsc

N = 1000000
B = 16384
D = 32
PACK = 128 // D
C = 1 << 18
CSHIFT = 18
NC = 2
NS = 16
L = 16
NW = NC * NS
BPW = B // NW
HALF = BPW // 2
G = HALF // L

BX = 1024
GRID = C // BX
NBLK = -(-N // BX) - 1


def _repack_body(t0, t1, t2, t3, out_ref, eye_scr):
    @pl.when(pl.program_id(0) == 0)
    def _init():
        r = lax.broadcasted_iota(jnp.int32, (128, 128), 0)
        c = lax.broadcasted_iota(jnp.int32, (128, 128), 1)
        eye_scr[...] = (r == c).astype(jnp.bfloat16)

    m = jnp.concatenate([t0[...], t1[...], t2[...], t3[...]], axis=0)
    out_ref[...] = lax.dot_general(
        m.astype(jnp.bfloat16), eye_scr[...], (((0,), (0,)), ((), ())),
        preferred_element_type=jnp.float32)


def _in_spec(k):
    return pl.BlockSpec(
        (D, BX), lambda g, k=k: (0, jnp.minimum(g + k * GRID, NBLK)))


_repack = pl.pallas_call(
    _repack_body,
    grid=(GRID,),
    in_specs=[_in_spec(0), _in_spec(1), _in_spec(2), _in_spec(3)],
    out_specs=pl.BlockSpec((BX, 128), lambda g: (g, 0)),
    out_shape=jax.ShapeDtypeStruct((C, 128), jnp.float32),
    scratch_shapes=[pltpu.VMEM((128, 128), jnp.bfloat16)],
)

_mesh = plsc.VectorSubcoreMesh(core_axis_name="c", subcore_axis_name="s")


@functools.partial(
    pl.kernel,
    mesh=_mesh,
    out_type=jax.ShapeDtypeStruct((B,), jnp.float32),
    scratch_types=[
        pltpu.VMEM((BPW,), jnp.int32),
        pltpu.VMEM((BPW,), jnp.int32),
        pltpu.VMEM((BPW,), jnp.int32),
        pltpu.VMEM((BPW,), jnp.int32),
        pltpu.VMEM((HALF, 128), jnp.float32),
        pltpu.VMEM((HALF, 128), jnp.float32),
        pltpu.VMEM((BPW,), jnp.float32),
        pltpu.SemaphoreType.DMA,
    ],
    compiler_params=pltpu.CompilerParams(
        needs_layout_passes=False, use_tc_tiling_on_sc=False),
)
def _mf_dot(u_hbm, v_hbm, up_hbm, vp_hbm, out_hbm,
            uraw, vraw, upk, vpk, rows_u, rows_v, outv, sem):
    wid = lax.axis_index("s") * NC + lax.axis_index("c")
    base = wid * BPW

    pltpu.sync_copy(u_hbm.at[pl.ds(base, BPW)], uraw)
    pltpu.sync_copy(v_hbm.at[pl.ds(base, BPW)], vraw)

    def build(t, carry):
        upk[pl.ds(t * L, L)] = uraw[pl.ds(t * L, L)] & (C - 1)
        vpk[pl.ds(t * L, L)] = vraw[pl.ds(t * L, L)] & (C - 1)
        return carry

    lax.fori_loop(0, BPW // L, build, 0)

    lane = lax.iota(jnp.int32, L)

    def half(h, carry):
        hb = h * HALF
        cu = pltpu.async_copy(up_hbm.at[upk.at[pl.ds(hb, HALF)]], rows_u, sem)
        cv = pltpu.async_copy(vp_hbm.at[vpk.at[pl.ds(hb, HALF)]], rows_v, sem)
        cu.wait()
        cv.wait()

        def group(g, carry2):
            row = g * L + lane
            offu = lax.shift_right_logical(uraw[pl.ds(hb + g * L, L)], CSHIFT) * D
            offv = lax.shift_right_logical(vraw[pl.ds(hb + g * L, L)], CSHIFT) * D
            acc = jnp.zeros((L,), jnp.float32)
            for d in range(D):
                xu = plsc.load_gather(rows_u, [row, offu + d])
                xv = plsc.load_gather(rows_v, [row, offv + d])
                acc = acc + xu * xv
            outv[pl.ds(hb + g * L, L)] = acc
            return carry2

        lax.fori_loop(0, G, group, 0)
        return carry

    lax.fori_loop(0, 2, half, 0)

    pltpu.sync_copy(outv, out_hbm.at[pl.ds(base, BPW)])


def kernel(u, v, user_emb, item_emb):
    ut = jnp.transpose(user_emb)
    vt = jnp.transpose(item_emb)
    up = _repack(ut, ut, ut, ut)
    vp = _repack(vt, vt, vt, vt)
    return _mf_dot(u.astype(jnp.int32), v.astype(jnp.int32), up, vp)


# speedup vs baseline: 2.2870x; 1.3965x over previous
import functools

import jax
import jax.numpy as jnp
from jax import lax
from jax.experimental import pallas as pl
from jax.experimental.pallas import tpu as pltpu
from jax.experimental.pallas import tpu_sc as plsc

N = 1000000
B = 16384
D = 32
PACK = 128 // D
C = 1 << 18
CSHIFT = 18
NC = 2
NS = 16
L = 16
NW = NC * NS
BPW = B // NW
HALF = BPW // 2
G = HALF // L

BX = 2048
GRID = C // BX
NBLK = -(-N // BX) - 1


def _repack_body(t0, t1, t2, t3, out_ref, eye_scr):
    @pl.when(pl.program_id(0) == 0)
    def _init():
        r = lax.broadcasted_iota(jnp.int32, (128, 128), 0)
        c = lax.broadcasted_iota(jnp.int32, (128, 128), 1)
        eye_scr[...] = (r == c).astype(jnp.bfloat16)

    m = jnp.concatenate([t0[...], t1[...], t2[...], t3[...]], axis=0)
    out_ref[...] = lax.dot_general(
        m.astype(jnp.bfloat16), eye_scr[...], (((0,), (0,)), ((), ())),
        preferred_element_type=jnp.float32)


def _in_spec(k):
    return pl.BlockSpec(
        (D, BX), lambda g, k=k: (0, jnp.minimum(g + k * GRID, NBLK)))


_repack = pl.pallas_call(
    _repack_body,
    grid=(GRID,),
    in_specs=[_in_spec(0), _in_spec(1), _in_spec(2), _in_spec(3)],
    out_specs=pl.BlockSpec((BX, 128), lambda g: (g, 0)),
    out_shape=jax.ShapeDtypeStruct((C, 128), jnp.float32),
    scratch_shapes=[pltpu.VMEM((128, 128), jnp.bfloat16)],
)

_mesh = plsc.VectorSubcoreMesh(core_axis_name="c", subcore_axis_name="s")


@functools.partial(
    pl.kernel,
    mesh=_mesh,
    out_type=jax.ShapeDtypeStruct((B,), jnp.float32),
    scratch_types=[
        pltpu.VMEM((BPW,), jnp.int32),
        pltpu.VMEM((BPW,), jnp.int32),
        pltpu.VMEM((BPW,), jnp.int32),
        pltpu.VMEM((BPW,), jnp.int32),
        pltpu.VMEM((HALF, 128), jnp.float32),
        pltpu.VMEM((HALF, 128), jnp.float32),
        pltpu.VMEM((BPW,), jnp.float32),
        pltpu.SemaphoreType.DMA,
    ],
    compiler_params=pltpu.CompilerParams(
        needs_layout_passes=False, use_tc_tiling_on_sc=False),
)
def _mf_dot(u_hbm, v_hbm, up_hbm, vp_hbm, out_hbm,
            uraw, vraw, upk, vpk, rows_u, rows_v, outv, sem):
    wid = lax.axis_index("s") * NC + lax.axis_index("c")
    base = wid * BPW

    pltpu.sync_copy(u_hbm.at[pl.ds(base, BPW)], uraw)
    pltpu.sync_copy(v_hbm.at[pl.ds(base, BPW)], vraw)

    def build(t, carry):
        upk[pl.ds(t * L, L)] = uraw[pl.ds(t * L, L)] & (C - 1)
        vpk[pl.ds(t * L, L)] = vraw[pl.ds(t * L, L)] & (C - 1)
        return carry

    lax.fori_loop(0, BPW // L, build, 0)

    lane = lax.iota(jnp.int32, L)

    def half(h, carry):
        hb = h * HALF
        cu = pltpu.async_copy(up_hbm.at[upk.at[pl.ds(hb, HALF)]], rows_u, sem)
        cv = pltpu.async_copy(vp_hbm.at[vpk.at[pl.ds(hb, HALF)]], rows_v, sem)
        cu.wait()
        cv.wait()

        def group(g, carry2):
            row = g * L + lane
            offu = lax.shift_right_logical(uraw[pl.ds(hb + g * L, L)], CSHIFT) * D
            offv = lax.shift_right_logical(vraw[pl.ds(hb + g * L, L)], CSHIFT) * D
            acc = jnp.zeros((L,), jnp.float32)
            for d in range(D):
                xu = plsc.load_gather(rows_u, [row, offu + d])
                xv = plsc.load_gather(rows_v, [row, offv + d])
                acc = acc + xu * xv
            outv[pl.ds(hb + g * L, L)] = acc
            return carry2

        lax.fori_loop(0, G, group, 0)
        return carry

    lax.fori_loop(0, 2, half, 0)

    pltpu.sync_copy(outv, out_hbm.at[pl.ds(base, BPW)])


def kernel(u, v, user_emb, item_emb):
    ut = jnp.transpose(user_emb)
    vt = jnp.transpose(item_emb)
    up = _repack(ut, ut, ut, ut)
    vp = _repack(vt, vt, vt, vt)
    return _mf_dot(u.astype(jnp.int32), v.astype(jnp.int32), up, vp)


# speedup vs baseline: 3.1171x; 1.3630x over previous
import functools

import jax
import jax.numpy as jnp
from jax import lax
from jax.experimental import pallas as pl
from jax.experimental.pallas import tpu as pltpu
from jax.experimental.pallas import tpu_sc as plsc

N = 1000000
B = 16384
D = 32
PACK = 128 // D
C = 1 << 18
CSHIFT = 18
NC = 2
NS = 16
L = 16
NW = NC * NS
BPW = B // NW
HALF = BPW // 2
G = HALF // L

BX = 4096
GRID = C // BX
NBLK = -(-N // BX) - 1


def _repack_body(t0, t1, t2, t3, out_ref, eye_scr):
    @pl.when(pl.program_id(0) == 0)
    def _init():
        r = lax.broadcasted_iota(jnp.int32, (128, 128), 0)
        c = lax.broadcasted_iota(jnp.int32, (128, 128), 1)
        eye_scr[...] = (r == c).astype(jnp.bfloat16)

    m = jnp.concatenate([t0[...], t1[...], t2[...], t3[...]], axis=0)
    out_ref[...] = lax.dot_general(
        m.astype(jnp.bfloat16), eye_scr[...], (((0,), (0,)), ((), ())),
        preferred_element_type=jnp.float32)


def _in_spec(k):
    return pl.BlockSpec(
        (D, BX), lambda g, k=k: (0, jnp.minimum(g + k * GRID, NBLK)))


_repack = pl.pallas_call(
    _repack_body,
    grid=(GRID,),
    in_specs=[_in_spec(0), _in_spec(1), _in_spec(2), _in_spec(3)],
    out_specs=pl.BlockSpec((BX, 128), lambda g: (g, 0)),
    out_shape=jax.ShapeDtypeStruct((C, 128), jnp.float32),
    scratch_shapes=[pltpu.VMEM((128, 128), jnp.bfloat16)],
)

_mesh = plsc.VectorSubcoreMesh(core_axis_name="c", subcore_axis_name="s")


@functools.partial(
    pl.kernel,
    mesh=_mesh,
    out_type=jax.ShapeDtypeStruct((B,), jnp.float32),
    scratch_types=[
        pltpu.VMEM((BPW,), jnp.int32),
        pltpu.VMEM((BPW,), jnp.int32),
        pltpu.VMEM((BPW,), jnp.int32),
        pltpu.VMEM((BPW,), jnp.int32),
        pltpu.VMEM((HALF, 128), jnp.float32),
        pltpu.VMEM((HALF, 128), jnp.float32),
        pltpu.VMEM((BPW,), jnp.float32),
        pltpu.SemaphoreType.DMA,
    ],
    compiler_params=pltpu.CompilerParams(
        needs_layout_passes=False, use_tc_tiling_on_sc=False),
)
def _mf_dot(u_hbm, v_hbm, up_hbm, vp_hbm, out_hbm,
            uraw, vraw, upk, vpk, rows_u, rows_v, outv, sem):
    wid = lax.axis_index("s") * NC + lax.axis_index("c")
    base = wid * BPW

    pltpu.sync_copy(u_hbm.at[pl.ds(base, BPW)], uraw)
    pltpu.sync_copy(v_hbm.at[pl.ds(base, BPW)], vraw)

    def build(t, carry):
        upk[pl.ds(t * L, L)] = uraw[pl.ds(t * L, L)] & (C - 1)
        vpk[pl.ds(t * L, L)] = vraw[pl.ds(t * L, L)] & (C - 1)
        return carry

    lax.fori_loop(0, BPW // L, build, 0)

    lane = lax.iota(jnp.int32, L)

    def half(h, carry):
        hb = h * HALF
        cu = pltpu.async_copy(up_hbm.at[upk.at[pl.ds(hb, HALF)]], rows_u, sem)
        cv = pltpu.async_copy(vp_hbm.at[vpk.at[pl.ds(hb, HALF)]], rows_v, sem)
        cu.wait()
        cv.wait()

        def group(g, carry2):
            row = g * L + lane
            offu = lax.shift_right_logical(uraw[pl.ds(hb + g * L, L)], CSHIFT) * D
            offv = lax.shift_right_logical(vraw[pl.ds(hb + g * L, L)], CSHIFT) * D
            acc = jnp.zeros((L,), jnp.float32)
            for d in range(D):
                xu = plsc.load_gather(rows_u, [row, offu + d])
                xv = plsc.load_gather(rows_v, [row, offv + d])
                acc = acc + xu * xv
            outv[pl.ds(hb + g * L, L)] = acc
            return carry2

        lax.fori_loop(0, G, group, 0)
        return carry

    lax.fori_loop(0, 2, half, 0)

    pltpu.sync_copy(outv, out_hbm.at[pl.ds(base, BPW)])


def kernel(u, v, user_emb, item_emb):
    ut = jnp.transpose(user_emb)
    vt = jnp.transpose(item_emb)
    up = _repack(ut, ut, ut, ut)
    vp = _repack(vt, vt, vt, vt)
    return _mf_dot(u.astype(jnp.int32), v.astype(jnp.int32), up, vp)


# speedup vs baseline: 3.5257x; 1.1311x over previous
import functools

import jax
import jax.numpy as jnp
from jax import lax
from jax.experimental import pallas as pl
from jax.experimental.pallas import tpu as pltpu
from jax.experimental.pallas import tpu_sc as plsc

N = 1000000
B = 16384
D = 32
PACK = 128 // D
C = 1 << 18
CSHIFT = 18
NC = 2
NS = 16
L = 16
NW = NC * NS
BPW = B // NW
HALF = BPW // 2
G = HALF // L

BX = 8192
GRID = C // BX
NBLK = -(-N // BX) - 1


def _repack_body(t0, t1, t2, t3, out_ref, eye_scr):
    @pl.when(pl.program_id(0) == 0)
    def _init():
        r = lax.broadcasted_iota(jnp.int32, (128, 128), 0)
        c = lax.broadcasted_iota(jnp.int32, (128, 128), 1)
        eye_scr[...] = (r == c).astype(jnp.bfloat16)

    m = jnp.concatenate([t0[...], t1[...], t2[...], t3[...]], axis=0)
    out_ref[...] = lax.dot_general(
        m.astype(jnp.bfloat16), eye_scr[...], (((0,), (0,)), ((), ())),
        preferred_element_type=jnp.float32)


def _in_spec(k):
    return pl.BlockSpec(
        (D, BX), lambda g, k=k: (0, jnp.minimum(g + k * GRID, NBLK)))


_repack = pl.pallas_call(
    _repack_body,
    grid=(GRID,),
    in_specs=[_in_spec(0), _in_spec(1), _in_spec(2), _in_spec(3)],
    out_specs=pl.BlockSpec((BX, 128), lambda g: (g, 0)),
    out_shape=jax.ShapeDtypeStruct((C, 128), jnp.float32),
    scratch_shapes=[pltpu.VMEM((128, 128), jnp.bfloat16)],
)

_mesh = plsc.VectorSubcoreMesh(core_axis_name="c", subcore_axis_name="s")


@functools.partial(
    pl.kernel,
    mesh=_mesh,
    out_type=jax.ShapeDtypeStruct((B,), jnp.float32),
    scratch_types=[
        pltpu.VMEM((BPW,), jnp.int32),
        pltpu.VMEM((BPW,), jnp.int32),
        pltpu.VMEM((BPW,), jnp.int32),
        pltpu.VMEM((BPW,), jnp.int32),
        pltpu.VMEM((HALF, 128), jnp.float32),
        pltpu.VMEM((HALF, 128), jnp.float32),
        pltpu.VMEM((BPW,), jnp.float32),
        pltpu.SemaphoreType.DMA,
    ],
    compiler_params=pltpu.CompilerParams(
        needs_layout_passes=False, use_tc_tiling_on_sc=False),
)
def _mf_dot(u_hbm, v_hbm, up_hbm, vp_hbm, out_hbm,
            uraw, vraw, upk, vpk, rows_u, rows_v, outv, sem):
    wid = lax.axis_index("s") * NC + lax.axis_index("c")
    base = wid * BPW

    pltpu.sync_copy(u_hbm.at[pl.ds(base, BPW)], uraw)
    pltpu.sync_copy(v_hbm.at[pl.ds(base, BPW)], vraw)

    def build(t, carry):
        upk[pl.ds(t * L, L)] = uraw[pl.ds(t * L, L)] & (C - 1)
        vpk[pl.ds(t * L, L)] = vraw[pl.ds(t * L, L)] & (C - 1)
        return carry

    lax.fori_loop(0, BPW // L, build, 0)

    lane = lax.iota(jnp.int32, L)

    def half(h, carry):
        hb = h * HALF
        cu = pltpu.async_copy(up_hbm.at[upk.at[pl.ds(hb, HALF)]], rows_u, sem)
        cv = pltpu.async_copy(vp_hbm.at[vpk.at[pl.ds(hb, HALF)]], rows_v, sem)
        cu.wait()
        cv.wait()

        def group(g, carry2):
            row = g * L + lane
            offu = lax.shift_right_logical(uraw[pl.ds(hb + g * L, L)], CSHIFT) * D
            offv = lax.shift_right_logical(vraw[pl.ds(hb + g * L, L)], CSHIFT) * D
            acc = jnp.zeros((L,), jnp.float32)
            for d in range(D):
                xu = plsc.load_gather(rows_u, [row, offu + d])
                xv = plsc.load_gather(rows_v, [row, offv + d])
                acc = acc + xu * xv
            outv[pl.ds(hb + g * L, L)] = acc
            return carry2

        lax.fori_loop(0, G, group, 0)
        return carry

    lax.fori_loop(0, 2, half, 0)

    pltpu.sync_copy(outv, out_hbm.at[pl.ds(base, BPW)])


def kernel(u, v, user_emb, item_emb):
    ut = jnp.transpose(user_emb)
    vt = jnp.transpose(item_emb)
    up = _repack(ut, ut, ut, ut)
    vp = _repack(vt, vt, vt, vt)
    return _mf_dot(u.astype(jnp.int32), v.astype(jnp.int32), up, vp)


# speedup vs baseline: 3.6178x; 1.0261x over previous
import functools

import jax
import jax.numpy as jnp
from jax import lax
from jax.experimental import pallas as pl
from jax.experimental.pallas import tpu as pltpu
from jax.experimental.pallas import tpu_sc as plsc

N = 1000000
B = 16384
D = 32
PACK = 128 // D
C = 1 << 18
CSHIFT = 18
NC = 2
NS = 16
L = 16
NW = NC * NS
BPW = B // NW
HALF = BPW // 2
G = HALF // L

BX = 16384
GRID = C // BX
NBLK = -(-N // BX) - 1


def _repack_body(t0, t1, t2, t3, out_ref, eye_scr):
    @pl.when(pl.program_id(0) == 0)
    def _init():
        r = lax.broadcasted_iota(jnp.int32, (128, 128), 0)
        c = lax.broadcasted_iota(jnp.int32, (128, 128), 1)
        eye_scr[...] = (r == c).astype(jnp.bfloat16)

    m = jnp.concatenate([t0[...], t1[...], t2[...], t3[...]], axis=0)
    out_ref[...] = lax.dot_general(
        m.astype(jnp.bfloat16), eye_scr[...], (((0,), (0,)), ((), ())),
        preferred_element_type=jnp.float32)


def _in_spec(k):
    return pl.BlockSpec(
        (D, BX), lambda g, k=k: (0, jnp.minimum(g + k * GRID, NBLK)))


_repack = pl.pallas_call(
    _repack_body,
    grid=(GRID,),
    in_specs=[_in_spec(0), _in_spec(1), _in_spec(2), _in_spec(3)],
    out_specs=pl.BlockSpec((BX, 128), lambda g: (g, 0)),
    out_shape=jax.ShapeDtypeStruct((C, 128), jnp.float32),
    scratch_shapes=[pltpu.VMEM((128, 128), jnp.bfloat16)],
)

_mesh = plsc.VectorSubcoreMesh(core_axis_name="c", subcore_axis_name="s")


@functools.partial(
    pl.kernel,
    mesh=_mesh,
    out_type=jax.ShapeDtypeStruct((B,), jnp.float32),
    scratch_types=[
        pltpu.VMEM((BPW,), jnp.int32),
        pltpu.VMEM((BPW,), jnp.int32),
        pltpu.VMEM((BPW,), jnp.int32),
        pltpu.VMEM((BPW,), jnp.int32),
        pltpu.VMEM((HALF, 128), jnp.float32),
        pltpu.VMEM((HALF, 128), jnp.float32),
        pltpu.VMEM((BPW,), jnp.float32),
        pltpu.SemaphoreType.DMA,
    ],
    compiler_params=pltpu.CompilerParams(
        needs_layout_passes=False, use_tc_tiling_on_sc=False),
)
def _mf_dot(u_hbm, v_hbm, up_hbm, vp_hbm, out_hbm,
            uraw, vraw, upk, vpk, rows_u, rows_v, outv, sem):
    wid = lax.axis_index("s") * NC + lax.axis_index("c")
    base = wid * BPW

    pltpu.sync_copy(u_hbm.at[pl.ds(base, BPW)], uraw)
    pltpu.sync_copy(v_hbm.at[pl.ds(base, BPW)], vraw)

    def build(t, carry):
        upk[pl.ds(t * L, L)] = uraw[pl.ds(t * L, L)] & (C - 1)
        vpk[pl.ds(t * L, L)] = vraw[pl.ds(t * L, L)] & (C - 1)
        return carry

    lax.fori_loop(0, BPW // L, build, 0)

    lane = lax.iota(jnp.int32, L)

    def half(h, carry):
        hb = h * HALF
        cu = pltpu.async_copy(up_hbm.at[upk.at[pl.ds(hb, HALF)]], rows_u, sem)
        cv = pltpu.async_copy(vp_hbm.at[vpk.at[pl.ds(hb, HALF)]], rows_v, sem)
        cu.wait()
        cv.wait()

        def group(g, carry2):
            row = g * L + lane
            offu = lax.shift_right_logical(uraw[pl.ds(hb + g * L, L)], CSHIFT) * D
            offv = lax.shift_right_logical(vraw[pl.ds(hb + g * L, L)], CSHIFT) * D
            acc = jnp.zeros((L,), jnp.float32)
            for d in range(D):
                xu = plsc.load_gather(rows_u, [row, offu + d])
                xv = plsc.load_gather(rows_v, [row, offv + d])
                acc = acc + xu * xv
            outv[pl.ds(hb + g * L, L)] = acc
            return carry2

        lax.fori_loop(0, G, group, 0)
        return carry

    lax.fori_loop(0, 2, half, 0)

    pltpu.sync_copy(outv, out_hbm.at[pl.ds(base, BPW)])


def kernel(u, v, user_emb, item_emb):
    ut = jnp.transpose(user_emb)
    vt = jnp.transpose(item_emb)
    up = _repack(ut, ut, ut, ut)
    vp = _repack(vt, vt, vt, vt)
    return _mf_dot(u.astype(jnp.int32), v.astype(jnp.int32), up, vp)
